# baseline (speedup 1.0000x reference)
import functools

import jax
import jax.numpy as jnp
from jax import lax
from jax.experimental import pallas as pl
from jax.experimental.pallas import tpu as pltpu
from jax.experimental.pallas import tpu_sc as plsc

_B, _S = 1024, 200
_L = 4
_VL, _VT = 100000, 512
_D = 32
_N = _B * _S
_NW = 32
_C = 104
_CP = 112
_BW = _B // _NW
_NCHW = 2 * _BW
_OFFS = (0, 16, 32, 48, 64, 80, 88)


def _make_kernel():
    mesh = plsc.VectorSubcoreMesh(core_axis_name="c", subcore_axis_name="s")

    @functools.partial(
        pl.kernel,
        mesh=mesh,
        out_type=jax.ShapeDtypeStruct((_B, _S, 2 * _D), jnp.float32),
        compiler_params=pltpu.CompilerParams(
            use_tc_tiling_on_sc=False, needs_layout_passes=False),
        scratch_types=[
            pltpu.VMEM((2, _C, _L), jnp.int32),
            pltpu.VMEM((2, _C, _L), jnp.int32),
            pltpu.VMEM((2, _L, _CP), jnp.int32),
            pltpu.VMEM((2, _L, _CP), jnp.int32),
            pltpu.VMEM((2, _L, _C, _D), jnp.float32),
            pltpu.VMEM((2, _L, _C, _D), jnp.float32),
            pltpu.VMEM((2, _C, 2 * _D), jnp.float32),
            pltpu.VMEM((2 * _L, 16), jnp.float32),
            pltpu.SemaphoreType.DMA,
            pltpu.SemaphoreType.DMA,
            pltpu.SemaphoreType.DMA,
            pltpu.SemaphoreType.DMA,
            pltpu.SemaphoreType.DMA,
            pltpu.SemaphoreType.DMA,
        ],
    )
    def k(x_hbm, t_hbm, loc_hbm, time_hbm, w_hbm, out_hbm,
          raw_x, raw_t, idx_x, idx_t, rows_x, rows_t, out_v, w_v,
          sem_i0, sem_i1, sem_g0, sem_g1, sem_o0, sem_o1):
        wid = lax.axis_index("s") * 2 + lax.axis_index("c")
        g0 = wid * _NCHW
        sem_i = [sem_i0, sem_i1]
        sem_g = [sem_g0, sem_g1]
        sem_o = [sem_o0, sem_o1]

        pltpu.sync_copy(w_hbm, w_v)
        ws = [w_v[j] for j in range(2 * _L)]
        rvecs = [lax.iota(jnp.int32, 16) + jnp.int32(o) for o in _OFFS]
        cvecs = [jnp.full((16,), l, jnp.int32) for l in range(_L)]

        def chunk_slices(g):
            return g // 2, (g % 2) * (_S - _C)

        def stage_idx(par, g):
            b, s0 = chunk_slices(g)
            pltpu.async_copy(x_hbm.at[b, pl.ds(s0, _C), :],
                             raw_x.at[par], sem_i[par])
            pltpu.async_copy(t_hbm.at[b, pl.ds(s0, _C), :],
                             raw_t.at[par], sem_i[par])

        def wait_idx(par):
            pltpu.make_async_copy(
                x_hbm.at[0, pl.ds(0, _C), :], raw_x.at[par],
                sem_i[par]).wait()
            pltpu.make_async_copy(
                t_hbm.at[0, pl.ds(0, _C), :], raw_t.at[par],
                sem_i[par]).wait()

        def transpose_idx(par):
            for l in range(_L):
                for o, rv in zip(_OFFS, rvecs):
                    idx_x[par, l, pl.ds(o, 16)] = plsc.load_gather(
                        raw_x.at[par], [rv, cvecs[l]])
                    idx_t[par, l, pl.ds(o, 16)] = plsc.load_gather(
                        raw_t.at[par], [rv, cvecs[l]])

        def issue_gathers(par):
            for l in range(_L):
                pltpu.async_copy(
                    loc_hbm.at[l].at[idx_x.at[par, l, pl.ds(0, _C)]],
                    rows_x.at[par, l], sem_g[par])
                pltpu.async_copy(
                    time_hbm.at[l].at[idx_t.at[par, l, pl.ds(0, _C)]],
                    rows_t.at[par, l], sem_g[par])

        def wait_gathers(par):
            for l in range(_L):
                pltpu.make_async_copy(
                    loc_hbm.at[0, pl.ds(0, _C), :], rows_x.at[par, l],
                    sem_g[par]).wait()
                pltpu.make_async_copy(
                    loc_hbm.at[0, pl.ds(0, _C), :], rows_t.at[par, l],
                    sem_g[par]).wait()

        def compute(par):
            @plsc.parallel_loop(0, _C, unroll=4)
            def _(c):
                for p in range(2):
                    sl = p * 16
                    a = ws[0] * rows_x[par, 0, c, pl.ds(sl, 16)]
                    for j in range(1, _L):
                        a = a + ws[j] * rows_x[par, j, c, pl.ds(sl, 16)]
                    out_v[par, c, pl.ds(sl, 16)] = a
                    b = ws[_L] * rows_t[par, 0, c, pl.ds(sl, 16)]
                    for j in range(1, _L):
                        b = b + ws[_L + j] * rows_t[par, j, c, pl.ds(sl, 16)]
                    out_v[par, c, pl.ds(_D + sl, 16)] = b

        def out_slice(g):
            b, s0 = chunk_slices(g)
            return out_hbm.at[b, pl.ds(s0, _C), :]

        stage_idx(0, g0)
        stage_idx(1, g0 + 1)
        wait_idx(0)
        transpose_idx(0)
        issue_gathers(0)

        def super_body(i, carry):
            for par in range(2):
                g = g0 + 2 * i + par
                nxt = 1 - par
                wait_gathers(par)

                @pl.when(i < _NCHW // 2 - 1)
                def _prefetch_idx():
                    stage_idx(par, g + 2)

                def _launch_next():
                    wait_idx(nxt)
                    transpose_idx(nxt)
                    issue_gathers(nxt)

                if par == 0:
                    _launch_next()
                else:
                    pl.when(i < _NCHW // 2 - 1)(_launch_next)

                @pl.when(i > 0)
                def _drain_out():
                    pltpu.make_async_copy(
                        out_v.at[par], out_slice(g - 2), sem_o[par]).wait()

                compute(par)
                pltpu.async_copy(out_v.at[par], out_slice(g), sem_o[par])
            return carry

        lax.fori_loop(0, _NCHW // 2, super_body, 0)

        last = g0 + _NCHW - 2
        pltpu.make_async_copy(out_v.at[0], out_slice(last), sem_o[0]).wait()
        pltpu.make_async_copy(out_v.at[1], out_slice(last + 1),
                              sem_o[1]).wait()

    return k


_k = _make_kernel()


def kernel(x, t, loc_tables, time_tables, x_weights, t_weights):
    w_all = jnp.broadcast_to(
        jnp.concatenate([x_weights, t_weights])[:, None], (2 * _L, 16))
    return _k(x.astype(jnp.int32), t.astype(jnp.int32),
              loc_tables, time_tables, w_all)
